# baseline (speedup 1.0000x reference)
import functools

import jax
import jax.numpy as jnp
from jax import lax
from jax.experimental import pallas as pl
from jax.experimental.pallas import tpu as pltpu
from jax.experimental.pallas import tpu_sc as plsc

NC = 2
NS = 16
NW = NC * NS
L = 16
NBUF = 10
WD = 5
CH = 8192
PAT = 1024


def _make_sc_add(NE, NP):
    per_w = NE // NW
    pat_w = NP // NW
    M = per_w // CH
    G = M // NBUF
    unit = 128 * PAT
    cpu = unit // CH
    assert NE % (NW * CH) == 0 and M % NBUF == 0 and unit % CH == 0

    mesh = plsc.VectorSubcoreMesh(
        core_axis_name="c", subcore_axis_name="s", num_cores=NC, num_subcores=NS
    )

    def compute(buf, pats, pat_off):
        def p_body(p, _):
            vp = pats[pl.ds(pat_off + p * L, L)]
            for r in range(CH // PAT):
                o = r * PAT
                buf[pl.ds(o + p * L, L)] = buf[pl.ds(o + p * L, L)] + vp
            return 0

        lax.fori_loop(0, PAT // L, p_body, 0)

    @functools.partial(
        pl.kernel,
        out_type=jax.ShapeDtypeStruct((NE,), jnp.float32),
        mesh=mesh,
        scratch_types=[
            pltpu.VMEM((pat_w,), jnp.float32),
            [pltpu.VMEM((CH,), jnp.float32)] * NBUF,
            [pltpu.SemaphoreType.DMA] * NBUF,
            [pltpu.SemaphoreType.DMA] * NBUF,
        ],
    )
    def sc_add(x_hbm, p_hbm, o_hbm, pats, bufs, isems, osems):
        wid = lax.axis_index("s") * NC + lax.axis_index("c")
        base = wid * per_w
        pltpu.sync_copy(p_hbm.at[pl.ds(wid * pat_w, pat_w)], pats)

        def in_copy(k, j):
            return pltpu.make_async_copy(
                x_hbm.at[pl.ds(base + k * CH, CH)], bufs[j], isems[j]
            )

        def out_copy(k, j):
            return pltpu.make_async_copy(
                bufs[j], o_hbm.at[pl.ds(base + k * CH, CH)], osems[j]
            )

        for j in range(NBUF - WD):
            in_copy(j, j).start()

        def body(K, _):
            c0 = K * NBUF
            for j in range(NBUF):
                k = c0 + j
                in_copy(k, j).wait()
                compute(bufs[j], pats, (k // cpu) * PAT)
                out_copy(k, j).start()

                jw = (j - WD) % NBUF

                @pl.when(k >= WD)
                def _():
                    out_copy(k - WD, jw).wait()

                @pl.when(k + NBUF - WD < M)
                def _():
                    in_copy(k + NBUF - WD, jw).start()

            return 0

        lax.fori_loop(0, G, body, 0)
        for t in range(WD):
            k = M - WD + t
            out_copy(k, k % NBUF).wait()

    return sc_add


def kernel(inputs, table):
    B, C, D = inputs.shape
    DT = D // 8
    BT = B // 128
    NE = B * C * D
    NP = C * DT * 8 * 128

    x5 = jnp.transpose(
        jnp.reshape(jnp.transpose(inputs, (1, 2, 0)), (C, DT, 8, BT, 128)),
        (0, 1, 3, 2, 4),
    )
    x1d = jnp.reshape(x5, (NE,))

    pat = jnp.reshape(
        jnp.broadcast_to(jnp.reshape(table, (C, DT, 8, 1)), (C, DT, 8, 128)), (NP,)
    )

    out1d = _make_sc_add(NE, NP)(x1d, pat)

    out5 = jnp.reshape(out1d, (C, DT, BT, 8, 128))
    out3 = jnp.reshape(jnp.transpose(out5, (0, 1, 3, 2, 4)), (C, D, B))
    return jnp.transpose(out3, (2, 0, 1))


# speedup vs baseline: 1.1912x; 1.1912x over previous
import functools

import jax
import jax.numpy as jnp
from jax import lax
from jax.experimental import pallas as pl
from jax.experimental.pallas import tpu as pltpu
from jax.experimental.pallas import tpu_sc as plsc

NC = 2
NS = 16
NW = NC * NS
L = 16
NBUF = 3
WD = 1
CH = 32768
PAT = 1024


def _make_sc_add(NE, NP):
    per_w = NE // NW
    pat_w = NP // NW
    M = per_w // CH
    G = M // NBUF
    unit = 128 * PAT
    cpu = unit // CH
    assert NE % (NW * CH) == 0 and unit % CH == 0

    mesh = plsc.VectorSubcoreMesh(
        core_axis_name="c", subcore_axis_name="s", num_cores=NC, num_subcores=NS
    )

    def compute(buf, pats, pat_off):
        def p_body(p, _):
            vp = pats[pl.ds(pat_off + p * L, L)]
            for r in range(CH // PAT):
                o = r * PAT
                buf[pl.ds(o + p * L, L)] = buf[pl.ds(o + p * L, L)] + vp
            return 0

        lax.fori_loop(0, PAT // L, p_body, 0)

    @functools.partial(
        pl.kernel,
        out_type=jax.ShapeDtypeStruct((NE,), jnp.float32),
        mesh=mesh,
        scratch_types=[
            pltpu.VMEM((pat_w,), jnp.float32),
            [pltpu.VMEM((CH,), jnp.float32)] * NBUF,
            [pltpu.SemaphoreType.DMA] * NBUF,
            [pltpu.SemaphoreType.DMA] * NBUF,
        ],
    )
    def sc_add(x_hbm, p_hbm, o_hbm, pats, bufs, isems, osems):
        wid = lax.axis_index("s") * NC + lax.axis_index("c")
        base = wid * per_w
        pltpu.sync_copy(p_hbm.at[pl.ds(wid * pat_w, pat_w)], pats)

        def in_copy(k, j):
            return pltpu.make_async_copy(
                x_hbm.at[pl.ds(base + k * CH, CH)], bufs[j], isems[j]
            )

        def out_copy(k, j):
            return pltpu.make_async_copy(
                bufs[j], o_hbm.at[pl.ds(base + k * CH, CH)], osems[j]
            )

        for j in range(NBUF - WD):
            in_copy(j, j).start()

        def body(K, _):
            c0 = K * NBUF
            for j in range(NBUF):
                k = c0 + j
                in_copy(k, j).wait()
                compute(bufs[j], pats, (k // cpu) * PAT)
                out_copy(k, j).start()

                jw = (j - WD) % NBUF

                @pl.when(k >= WD)
                def _():
                    out_copy(k - WD, jw).wait()

                @pl.when(k + NBUF - WD < M)
                def _():
                    in_copy(k + NBUF - WD, jw).start()

            return 0

        lax.fori_loop(0, G, body, 0)
        for k in range(G * NBUF, M):
            j = k % NBUF
            in_copy(k, j).wait()
            compute(bufs[j], pats, (k // cpu) * PAT)
            out_copy(k, j).start()
            out_copy(k - WD, (j - WD) % NBUF).wait()
        for t in range(WD):
            k = M - WD + t
            out_copy(k, k % NBUF).wait()

    return sc_add


def kernel(inputs, table):
    B, C, D = inputs.shape
    DT = D // 8
    BT = B // 128
    NE = B * C * D
    NP = C * DT * 8 * 128

    x5 = jnp.transpose(
        jnp.reshape(jnp.transpose(inputs, (1, 2, 0)), (C, DT, 8, BT, 128)),
        (0, 1, 3, 2, 4),
    )
    x1d = jnp.reshape(x5, (NE,))

    pat = jnp.reshape(
        jnp.broadcast_to(jnp.reshape(table, (C, DT, 8, 1)), (C, DT, 8, 128)), (NP,)
    )

    out1d = _make_sc_add(NE, NP)(x1d, pat)

    out5 = jnp.reshape(out1d, (C, DT, BT, 8, 128))
    out3 = jnp.reshape(jnp.transpose(out5, (0, 1, 3, 2, 4)), (C, D, B))
    return jnp.transpose(out3, (2, 0, 1))
